# Initial kernel scaffold; baseline (speedup 1.0000x reference)
import jax
import jax.numpy as jnp
from jax.experimental import pallas as pl


def kernel(query, value, W_v, b_v, W_off, b_off, W_attn, b_attn, W_out1, b_out1, W_out2, b_out2):
    raise NotImplementedError("write your pallas kernel here")


# speedup vs baseline: 27.5490x; 27.5490x over previous
import functools

import jax
import jax.numpy as jnp
from jax import lax
from jax.experimental import pallas as pl
from jax.experimental.pallas import tpu as pltpu
from jax.experimental.pallas import tpu_sc as plsc

B, C, NQ = 4, 256, 4096
HEADS, P = 8, 4
TAPS = 16
QB = 512
SQ = 512
NW = 32
NTASK = B * HEADS * 2


def _prep_body(qT_ref, vT_ref, wvt_ref, bv_ref, wofft_ref, boff_ref,
               wattnt_ref, battn_ref, vp_ref, idx_ref, w_ref):
    qi = pl.program_id(1)
    qT = qT_ref[0]
    vT = vT_ref[0]

    vp = jnp.dot(wvt_ref[...], vT, preferred_element_type=jnp.float32)
    vp_ref[0] = vp + bv_ref[...]

    off = jnp.dot(wofft_ref[...], qT, preferred_element_type=jnp.float32)
    off = (off + boff_ref[...]).reshape(HEADS, P, 2, QB)
    logits = jnp.dot(wattnt_ref[...], qT, preferred_element_type=jnp.float32)
    logits = (logits + battn_ref[...]).reshape(HEADS, P, QB)
    logits = logits - jnp.max(logits, axis=1, keepdims=True)
    e = jnp.exp(logits)
    attn = e / jnp.sum(e, axis=1, keepdims=True)

    lane = lax.broadcasted_iota(jnp.int32, (1, 1, QB), 2) + qi * QB
    rx = (lane // 64).astype(jnp.float32) * (64.0 / 63.0)
    ry = (lane % 64).astype(jnp.float32) * (64.0 / 63.0)

    x = rx + off[:, :, 0, :] - 0.5
    y = ry + off[:, :, 1, :] - 0.5
    x0 = jnp.floor(x)
    y0 = jnp.floor(y)
    x1 = x0 + 1.0
    y1 = y0 + 1.0
    fx = x - x0
    fy = y - y0
    vx0 = (x0 >= 0.0) & (x0 <= 63.0)
    vx1 = (x1 >= 0.0) & (x1 <= 63.0)
    vy0 = (y0 >= 0.0) & (y0 <= 63.0)
    vy1 = (y1 >= 0.0) & (y1 <= 63.0)
    cx0 = jnp.clip(x0, 0.0, 63.0).astype(jnp.int32)
    cx1 = jnp.clip(x1, 0.0, 63.0).astype(jnp.int32)
    cy0 = jnp.clip(y0, 0.0, 63.0).astype(jnp.int32) * 64
    cy1 = jnp.clip(y1, 0.0, 63.0).astype(jnp.int32) * 64

    wa = attn * ((1.0 - fx) * (1.0 - fy)) * (vx0 & vy0).astype(jnp.float32)
    wb = attn * ((1.0 - fx) * fy) * (vx0 & vy1).astype(jnp.float32)
    wc = attn * (fx * (1.0 - fy)) * (vx1 & vy0).astype(jnp.float32)
    wd = attn * (fx * fy) * (vx1 & vy1).astype(jnp.float32)

    ia = cy0 + cx0
    ib = cy1 + cx0
    ic = cy0 + cx1
    idd = cy1 + cx1

    w_ref[0] = jnp.stack([wa, wb, wc, wd], axis=2).reshape(HEADS * TAPS, QB)
    idx_ref[0] = jnp.stack([ia, ib, ic, idd], axis=2).reshape(HEADS * TAPS, QB)


_prep_call = pl.pallas_call(
    _prep_body,
    grid=(B, NQ // QB),
    in_specs=[
        pl.BlockSpec((1, C, QB), lambda b, q: (b, 0, q)),
        pl.BlockSpec((1, C, QB), lambda b, q: (b, 0, q)),
        pl.BlockSpec((C, C), lambda b, q: (0, 0)),
        pl.BlockSpec((C, 1), lambda b, q: (0, 0)),
        pl.BlockSpec((HEADS * P * 2, C), lambda b, q: (0, 0)),
        pl.BlockSpec((HEADS * P * 2, 1), lambda b, q: (0, 0)),
        pl.BlockSpec((HEADS * P, C), lambda b, q: (0, 0)),
        pl.BlockSpec((HEADS * P, 1), lambda b, q: (0, 0)),
    ],
    out_specs=[
        pl.BlockSpec((1, C, QB), lambda b, q: (b, 0, q)),
        pl.BlockSpec((1, HEADS * TAPS, QB), lambda b, q: (b, 0, q)),
        pl.BlockSpec((1, HEADS * TAPS, QB), lambda b, q: (b, 0, q)),
    ],
    out_shape=[
        jax.ShapeDtypeStruct((B, C, NQ), jnp.float32),
        jax.ShapeDtypeStruct((B, HEADS * TAPS, NQ), jnp.int32),
        jax.ShapeDtypeStruct((B, HEADS * TAPS, NQ), jnp.float32),
    ],
)


def _sc_body(vp_hbm, idx_hbm, w_hbm, out_hbm, img_v, idx_v, w_v, out_v):
    wid = lax.axis_index("s") * 2 + lax.axis_index("c")
    for r in range(NTASK // NW):
        task = wid + NW * r
        bh = task // 2
        pltpu.sync_copy(vp_hbm.at[task], img_v)
        for sb in range(NQ // SQ):
            qs = sb * SQ
            pltpu.sync_copy(idx_hbm.at[bh, :, pl.ds(qs, SQ)], idx_v)
            pltpu.sync_copy(w_hbm.at[bh, :, pl.ds(qs, SQ)], w_v)

            def group(g, carry):
                qo = g * 16
                accs = [jnp.zeros((16,), jnp.float32) for _ in range(16)]
                for t in range(TAPS):
                    it = idx_v[t, pl.ds(qo, 16)]
                    wt = w_v[t, pl.ds(qo, 16)]
                    for c in range(16):
                        gv = plsc.load_gather(img_v, [it + jnp.int32(c * NQ)])
                        accs[c] = accs[c] + wt * gv
                for c in range(16):
                    out_v[c, pl.ds(qo, 16)] = accs[c]
                return carry

            lax.fori_loop(0, SQ // 16, group, 0)
            pltpu.sync_copy(out_v, out_hbm.at[task, :, pl.ds(qs, SQ)])


_sc_call = functools.partial(
    pl.kernel,
    mesh=plsc.VectorSubcoreMesh(core_axis_name="c", subcore_axis_name="s"),
    compiler_params=pltpu.CompilerParams(use_tc_tiling_on_sc=False,
                                         needs_layout_passes=False),
    out_type=jax.ShapeDtypeStruct((NTASK, 16, NQ), jnp.float32),
    scratch_types=[
        pltpu.VMEM((16 * NQ,), jnp.float32),
        pltpu.VMEM((TAPS, SQ), jnp.int32),
        pltpu.VMEM((TAPS, SQ), jnp.float32),
        pltpu.VMEM((16, SQ), jnp.float32),
    ],
)(_sc_body)


def _out_body(msdaT_ref, qT_ref, vT_ref, w1t_ref, b1_ref, w2t_ref, b2_ref, o_ref):
    m1 = jnp.dot(w1t_ref[...], msdaT_ref[0], preferred_element_type=jnp.float32)
    m1 = m1 + b1_ref[...] + qT_ref[0]
    o = jnp.dot(w2t_ref[...], m1, preferred_element_type=jnp.float32)
    o_ref[0] = o + b2_ref[...] + vT_ref[0]


_out_call = pl.pallas_call(
    _out_body,
    grid=(B, NQ // QB),
    in_specs=[
        pl.BlockSpec((1, C, QB), lambda b, q: (b, 0, q)),
        pl.BlockSpec((1, C, QB), lambda b, q: (b, 0, q)),
        pl.BlockSpec((1, C, QB), lambda b, q: (b, 0, q)),
        pl.BlockSpec((C, C), lambda b, q: (0, 0)),
        pl.BlockSpec((C, 1), lambda b, q: (0, 0)),
        pl.BlockSpec((C, C), lambda b, q: (0, 0)),
        pl.BlockSpec((C, 1), lambda b, q: (0, 0)),
    ],
    out_specs=pl.BlockSpec((1, C, QB), lambda b, q: (b, 0, q)),
    out_shape=jax.ShapeDtypeStruct((B, C, NQ), jnp.float32),
)


def kernel(query, value, W_v, b_v, W_off, b_off, W_attn, b_attn,
           W_out1, b_out1, W_out2, b_out2):
    qT = query.reshape(B, C, NQ)
    vT = value.reshape(B, C, NQ)
    vp, idx, wts = _prep_call(
        qT, vT, W_v.T, b_v.reshape(C, 1), W_off.T, b_off.reshape(-1, 1),
        W_attn.T, b_attn.reshape(-1, 1))
    msdaT = _sc_call(vp.reshape(NTASK, 16 * NQ),
                     idx.reshape(B * HEADS, TAPS, NQ),
                     wts.reshape(B * HEADS, TAPS, NQ))
    outT = _out_call(msdaT.reshape(B, C, NQ), qT, vT,
                     W_out1.T, b_out1.reshape(C, 1),
                     W_out2.T, b_out2.reshape(C, 1))
    return outT.reshape(B, C, 64, 64)


# speedup vs baseline: 56.1979x; 2.0399x over previous
import functools

import jax
import jax.numpy as jnp
from jax import lax
from jax.experimental import pallas as pl
from jax.experimental.pallas import tpu as pltpu
from jax.experimental.pallas import tpu_sc as plsc

B, C, NQ = 4, 256, 4096
HEADS, P = 8, 4
TAPS = 16
QB = 512
SQ = 512
UNROLL = 4
NW = 32
NTASK = B * HEADS * 2


def _prep_body(qT_ref, vT_ref, wvt_ref, bv_ref, wofft_ref, boff_ref,
               wattnt_ref, battn_ref, vp_ref, pk_ref):
    qi = pl.program_id(1)
    qT = qT_ref[0]
    vT = vT_ref[0]

    vp = jnp.dot(wvt_ref[...], vT, preferred_element_type=jnp.float32)
    vp_ref[0] = vp + bv_ref[...]

    off = jnp.dot(wofft_ref[...], qT, preferred_element_type=jnp.float32)
    off = (off + boff_ref[...]).reshape(HEADS, P, 2, QB)
    logits = jnp.dot(wattnt_ref[...], qT, preferred_element_type=jnp.float32)
    logits = (logits + battn_ref[...]).reshape(HEADS, P, QB)
    logits = logits - jnp.max(logits, axis=1, keepdims=True)
    e = jnp.exp(logits)
    attn = e / jnp.sum(e, axis=1, keepdims=True)

    lane = lax.broadcasted_iota(jnp.int32, (1, 1, QB), 2) + qi * QB
    rx = (lane // 64).astype(jnp.float32) * (64.0 / 63.0)
    ry = (lane % 64).astype(jnp.float32) * (64.0 / 63.0)

    x = rx + off[:, :, 0, :] - 0.5
    y = ry + off[:, :, 1, :] - 0.5
    x0 = jnp.floor(x)
    y0 = jnp.floor(y)
    x1 = x0 + 1.0
    y1 = y0 + 1.0
    fx = x - x0
    fy = y - y0
    vx0 = (x0 >= 0.0) & (x0 <= 63.0)
    vx1 = (x1 >= 0.0) & (x1 <= 63.0)
    vy0 = (y0 >= 0.0) & (y0 <= 63.0)
    vy1 = (y1 >= 0.0) & (y1 <= 63.0)
    cx0 = jnp.clip(x0, 0.0, 63.0).astype(jnp.int32) * 16
    cx1 = jnp.clip(x1, 0.0, 63.0).astype(jnp.int32) * 16
    cy0 = jnp.clip(y0, 0.0, 63.0).astype(jnp.int32) * 1024
    cy1 = jnp.clip(y1, 0.0, 63.0).astype(jnp.int32) * 1024

    wa = attn * ((1.0 - fx) * (1.0 - fy)) * (vx0 & vy0).astype(jnp.float32)
    wb = attn * ((1.0 - fx) * fy) * (vx0 & vy1).astype(jnp.float32)
    wc = attn * (fx * (1.0 - fy)) * (vx1 & vy0).astype(jnp.float32)
    wd = attn * (fx * fy) * (vx1 & vy1).astype(jnp.float32)

    hi = jnp.int32(-65536)
    pa = (lax.bitcast_convert_type(wa, jnp.int32) & hi) | (cy0 + cx0)
    pb = (lax.bitcast_convert_type(wb, jnp.int32) & hi) | (cy1 + cx0)
    pc = (lax.bitcast_convert_type(wc, jnp.int32) & hi) | (cy0 + cx1)
    pd = (lax.bitcast_convert_type(wd, jnp.int32) & hi) | (cy1 + cx1)

    pk_ref[0] = jnp.stack([pa, pb, pc, pd], axis=2).reshape(HEADS * TAPS, QB)


_prep_call = pl.pallas_call(
    _prep_body,
    grid=(B, NQ // QB),
    in_specs=[
        pl.BlockSpec((1, C, QB), lambda b, q: (b, 0, q)),
        pl.BlockSpec((1, C, QB), lambda b, q: (b, 0, q)),
        pl.BlockSpec((C, C), lambda b, q: (0, 0)),
        pl.BlockSpec((C, 1), lambda b, q: (0, 0)),
        pl.BlockSpec((HEADS * P * 2, C), lambda b, q: (0, 0)),
        pl.BlockSpec((HEADS * P * 2, 1), lambda b, q: (0, 0)),
        pl.BlockSpec((HEADS * P, C), lambda b, q: (0, 0)),
        pl.BlockSpec((HEADS * P, 1), lambda b, q: (0, 0)),
    ],
    out_specs=[
        pl.BlockSpec((1, C, QB), lambda b, q: (b, 0, q)),
        pl.BlockSpec((1, HEADS * TAPS, QB), lambda b, q: (b, 0, q)),
    ],
    out_shape=[
        jax.ShapeDtypeStruct((B, C, NQ), jnp.float32),
        jax.ShapeDtypeStruct((B, HEADS * TAPS, NQ), jnp.int32),
    ],
)


def _sc_body(img_hbm, pk_hbm, out_hbm, img_v, pk_v, out_v):
    wid = lax.axis_index("s") * 2 + lax.axis_index("c")
    iota = lax.iota(jnp.int32, 16)
    for r in range(NTASK // NW):
        task = wid + NW * r
        bh = task // 2
        pltpu.sync_copy(img_hbm.at[task], img_v)
        for sb in range(NQ // SQ):
            qs = sb * SQ * 16
            pltpu.sync_copy(pk_hbm.at[bh, pl.ds(qs, SQ * 16)], pk_v)

            _dn = lax.GatherDimensionNumbers(
                offset_dims=(), collapsed_slice_dims=(0,), start_index_map=(0,))

            def qbody(g, carry):
                for u in range(UNROLL):
                    qo = (g * UNROLL + u) * 16
                    pk = pk_v[pl.ds(qo, 16)]
                    acc = jnp.zeros((16,), jnp.float32)
                    for t in range(TAPS):
                        sp = lax.gather(
                            pk, jnp.full((16, 1), t, jnp.int32), _dn,
                            slice_sizes=(1,),
                            mode=lax.GatherScatterMode.PROMISE_IN_BOUNDS)
                        w = plsc.bitcast(sp, jnp.float32)
                        ib = (sp & jnp.int32(0xFFFF)) + iota
                        gv = plsc.load_gather(img_v, [ib])
                        acc = acc + w * gv
                    out_v[pl.ds(qo, 16)] = acc
                return carry

            lax.fori_loop(0, SQ // UNROLL, qbody, 0)
            pltpu.sync_copy(out_v, out_hbm.at[task, pl.ds(qs, SQ * 16)])


_sc_call = functools.partial(
    pl.kernel,
    mesh=plsc.VectorSubcoreMesh(core_axis_name="c", subcore_axis_name="s"),
    compiler_params=pltpu.CompilerParams(use_tc_tiling_on_sc=False,
                                         needs_layout_passes=False),
    out_type=jax.ShapeDtypeStruct((NTASK, 16 * NQ), jnp.float32),
    scratch_types=[
        pltpu.VMEM((16 * NQ,), jnp.float32),
        pltpu.VMEM((SQ * 16,), jnp.int32),
        pltpu.VMEM((SQ * 16,), jnp.float32),
    ],
)(_sc_body)


def _out_body(msdaT_ref, qT_ref, vT_ref, w1t_ref, b1_ref, w2t_ref, b2_ref, o_ref):
    m1 = jnp.dot(w1t_ref[...], msdaT_ref[0], preferred_element_type=jnp.float32)
    m1 = m1 + b1_ref[...] + qT_ref[0]
    o = jnp.dot(w2t_ref[...], m1, preferred_element_type=jnp.float32)
    o_ref[0] = o + b2_ref[...] + vT_ref[0]


_out_call = pl.pallas_call(
    _out_body,
    grid=(B, NQ // QB),
    in_specs=[
        pl.BlockSpec((1, C, QB), lambda b, q: (b, 0, q)),
        pl.BlockSpec((1, C, QB), lambda b, q: (b, 0, q)),
        pl.BlockSpec((1, C, QB), lambda b, q: (b, 0, q)),
        pl.BlockSpec((C, C), lambda b, q: (0, 0)),
        pl.BlockSpec((C, 1), lambda b, q: (0, 0)),
        pl.BlockSpec((C, C), lambda b, q: (0, 0)),
        pl.BlockSpec((C, 1), lambda b, q: (0, 0)),
    ],
    out_specs=pl.BlockSpec((1, C, QB), lambda b, q: (b, 0, q)),
    out_shape=jax.ShapeDtypeStruct((B, C, NQ), jnp.float32),
)


def kernel(query, value, W_v, b_v, W_off, b_off, W_attn, b_attn,
           W_out1, b_out1, W_out2, b_out2):
    qT = query.reshape(B, C, NQ)
    vT = value.reshape(B, C, NQ)
    vp, pk = _prep_call(
        qT, vT, W_v.T, b_v.reshape(C, 1), W_off.T, b_off.reshape(-1, 1),
        W_attn.T, b_attn.reshape(-1, 1))
    img = vp.reshape(B, HEADS, 2, 16, NQ).transpose(0, 1, 2, 4, 3)
    img = img.reshape(NTASK, 16 * NQ)
    pkq = pk.reshape(B, HEADS, TAPS, NQ).transpose(0, 1, 3, 2)
    pkq = pkq.reshape(B * HEADS, NQ * TAPS)
    smp = _sc_call(img, pkq)
    msdaT = smp.reshape(B, HEADS, 2, NQ, 16).transpose(0, 1, 2, 4, 3)
    msdaT = msdaT.reshape(B, C, NQ)
    outT = _out_call(msdaT, qT, vT,
                     W_out1.T, b_out1.reshape(C, 1),
                     W_out2.T, b_out2.reshape(C, 1))
    return outT.reshape(B, C, 64, 64)


# speedup vs baseline: 68.1395x; 1.2125x over previous
import functools

import jax
import jax.numpy as jnp
from jax import lax
from jax.experimental import pallas as pl
from jax.experimental.pallas import tpu as pltpu
from jax.experimental.pallas import tpu_sc as plsc

B, C, NQ = 4, 256, 4096
HEADS, P = 8, 4
TAPS = 16
QB = 512
SQ = 512
UNROLL = 4
NW = 32
NTASK = B * HEADS * 2


def _prep_body(qT_ref, vT_ref, wvt_ref, bv_ref, wofft_ref, boff_ref,
               wattnt_ref, battn_ref, vp_ref, pk_ref):
    qi = pl.program_id(1)
    qT = qT_ref[0]
    vT = vT_ref[0]

    vp = jnp.dot(wvt_ref[...], vT, preferred_element_type=jnp.float32)
    vp_ref[0] = jnp.transpose(vp + bv_ref[...], (1, 0))

    off = jnp.dot(wofft_ref[...], qT, preferred_element_type=jnp.float32)
    off = (off + boff_ref[...]).reshape(HEADS, P, 2, QB)
    logits = jnp.dot(wattnt_ref[...], qT, preferred_element_type=jnp.float32)
    logits = (logits + battn_ref[...]).reshape(HEADS, P, QB)
    logits = logits - jnp.max(logits, axis=1, keepdims=True)
    e = jnp.exp(logits)
    attn = e / jnp.sum(e, axis=1, keepdims=True)

    lane = lax.broadcasted_iota(jnp.int32, (1, 1, QB), 2) + qi * QB
    rx = (lane // 64).astype(jnp.float32) * (64.0 / 63.0)
    ry = (lane % 64).astype(jnp.float32) * (64.0 / 63.0)

    x = rx + off[:, :, 0, :] - 0.5
    y = ry + off[:, :, 1, :] - 0.5
    x0 = jnp.floor(x)
    y0 = jnp.floor(y)
    x1 = x0 + 1.0
    y1 = y0 + 1.0
    fx = x - x0
    fy = y - y0
    vx0 = (x0 >= 0.0) & (x0 <= 63.0)
    vx1 = (x1 >= 0.0) & (x1 <= 63.0)
    vy0 = (y0 >= 0.0) & (y0 <= 63.0)
    vy1 = (y1 >= 0.0) & (y1 <= 63.0)
    cx0 = jnp.clip(x0, 0.0, 63.0).astype(jnp.int32)
    cx1 = jnp.clip(x1, 0.0, 63.0).astype(jnp.int32)
    cy0 = jnp.clip(y0, 0.0, 63.0).astype(jnp.int32) * 64
    cy1 = jnp.clip(y1, 0.0, 63.0).astype(jnp.int32) * 64

    wa = attn * ((1.0 - fx) * (1.0 - fy)) * (vx0 & vy0).astype(jnp.float32)
    wb = attn * ((1.0 - fx) * fy) * (vx0 & vy1).astype(jnp.float32)
    wc = attn * (fx * (1.0 - fy)) * (vx1 & vy0).astype(jnp.float32)
    wd = attn * (fx * fy) * (vx1 & vy1).astype(jnp.float32)

    hi = jnp.int32(-65536)
    pa = (lax.bitcast_convert_type(wa, jnp.int32) & hi) | (cy0 + cx0)
    pb = (lax.bitcast_convert_type(wb, jnp.int32) & hi) | (cy1 + cx0)
    pc = (lax.bitcast_convert_type(wc, jnp.int32) & hi) | (cy0 + cx1)
    pd = (lax.bitcast_convert_type(wd, jnp.int32) & hi) | (cy1 + cx1)

    pk = jnp.stack([pa, pb, pc, pd], axis=2).reshape(HEADS * TAPS, QB)
    pk_ref[0] = jnp.transpose(pk, (1, 0))


_prep_call = pl.pallas_call(
    _prep_body,
    grid=(B, NQ // QB),
    in_specs=[
        pl.BlockSpec((1, C, QB), lambda b, q: (b, 0, q)),
        pl.BlockSpec((1, C, QB), lambda b, q: (b, 0, q)),
        pl.BlockSpec((C, C), lambda b, q: (0, 0)),
        pl.BlockSpec((C, 1), lambda b, q: (0, 0)),
        pl.BlockSpec((HEADS * P * 2, C), lambda b, q: (0, 0)),
        pl.BlockSpec((HEADS * P * 2, 1), lambda b, q: (0, 0)),
        pl.BlockSpec((HEADS * P, C), lambda b, q: (0, 0)),
        pl.BlockSpec((HEADS * P, 1), lambda b, q: (0, 0)),
    ],
    out_specs=[
        pl.BlockSpec((1, QB, C), lambda b, q: (b, q, 0)),
        pl.BlockSpec((1, QB, HEADS * TAPS), lambda b, q: (b, q, 0)),
    ],
    out_shape=[
        jax.ShapeDtypeStruct((B, NQ, C), jnp.float32),
        jax.ShapeDtypeStruct((B, NQ, HEADS * TAPS), jnp.int32),
    ],
)


def _sc_body(vp_hbm, pk_hbm, out_hbm, img_v, pk_v, out_v):
    wid = lax.axis_index("s") * 2 + lax.axis_index("c")
    iota = lax.iota(jnp.int32, 16)
    _dn = lax.GatherDimensionNumbers(
        offset_dims=(), collapsed_slice_dims=(0,), start_index_map=(0,))
    for r in range(NTASK // NW):
        task = wid + NW * r
        b = task // 16
        hh = task % 16
        pltpu.sync_copy(vp_hbm.at[b, :, pl.ds(hh * 16, 16)], img_v)
        for sb in range(NQ // SQ):
            qs = sb * SQ
            pltpu.sync_copy(
                pk_hbm.at[b, pl.ds(qs, SQ), pl.ds((hh // 2) * TAPS, TAPS)],
                pk_v)

            def qbody(g, carry):
                for u in range(UNROLL):
                    qo = g * UNROLL + u
                    pk = pk_v[qo]
                    acc = jnp.zeros((16,), jnp.float32)
                    for t in range(TAPS):
                        sp = lax.gather(
                            pk, jnp.full((16, 1), t, jnp.int32), _dn,
                            slice_sizes=(1,),
                            mode=lax.GatherScatterMode.PROMISE_IN_BOUNDS)
                        w = plsc.bitcast(sp, jnp.float32)
                        sv = sp & jnp.int32(0xFFFF)
                        gv = plsc.load_gather(img_v, [sv, iota])
                        acc = acc + w * gv
                    out_v[qo] = acc
                return carry

            lax.fori_loop(0, SQ // UNROLL, qbody, 0)
            pltpu.sync_copy(out_v, out_hbm.at[b, hh, pl.ds(qs, SQ), :])


_sc_call = functools.partial(
    pl.kernel,
    mesh=plsc.VectorSubcoreMesh(core_axis_name="c", subcore_axis_name="s"),
    compiler_params=pltpu.CompilerParams(use_tc_tiling_on_sc=False,
                                         needs_layout_passes=False),
    out_type=jax.ShapeDtypeStruct((B, 16, NQ, 16), jnp.float32),
    scratch_types=[
        pltpu.VMEM((NQ, 16), jnp.float32),
        pltpu.VMEM((SQ, TAPS), jnp.int32),
        pltpu.VMEM((SQ, 16), jnp.float32),
    ],
)(_sc_body)


def _out_body(msda_ref, qT_ref, vT_ref, w1t_ref, b1_ref, w2t_ref, b2_ref, o_ref):
    m = msda_ref[0]
    msdaT = jnp.transpose(m, (0, 2, 1)).reshape(C, QB)
    m1 = jnp.dot(w1t_ref[...], msdaT, preferred_element_type=jnp.float32)
    m1 = m1 + b1_ref[...] + qT_ref[0]
    o = jnp.dot(w2t_ref[...], m1, preferred_element_type=jnp.float32)
    o_ref[0] = o + b2_ref[...] + vT_ref[0]


_out_call = pl.pallas_call(
    _out_body,
    grid=(B, NQ // QB),
    in_specs=[
        pl.BlockSpec((1, 16, QB, 16), lambda b, q: (b, 0, q, 0)),
        pl.BlockSpec((1, C, QB), lambda b, q: (b, 0, q)),
        pl.BlockSpec((1, C, QB), lambda b, q: (b, 0, q)),
        pl.BlockSpec((C, C), lambda b, q: (0, 0)),
        pl.BlockSpec((C, 1), lambda b, q: (0, 0)),
        pl.BlockSpec((C, C), lambda b, q: (0, 0)),
        pl.BlockSpec((C, 1), lambda b, q: (0, 0)),
    ],
    out_specs=pl.BlockSpec((1, C, QB), lambda b, q: (b, 0, q)),
    out_shape=jax.ShapeDtypeStruct((B, C, NQ), jnp.float32),
)


def kernel(query, value, W_v, b_v, W_off, b_off, W_attn, b_attn,
           W_out1, b_out1, W_out2, b_out2):
    qT = query.reshape(B, C, NQ)
    vT = value.reshape(B, C, NQ)
    vp, pk = _prep_call(
        qT, vT, W_v.T, b_v.reshape(C, 1), W_off.T, b_off.reshape(-1, 1),
        W_attn.T, b_attn.reshape(-1, 1))
    msda = _sc_call(vp, pk)
    outT = _out_call(msda, qT, vT,
                     W_out1.T, b_out1.reshape(C, 1),
                     W_out2.T, b_out2.reshape(C, 1))
    return outT.reshape(B, C, 64, 64)
